# baseline (speedup 1.0000x reference)
import functools
import numpy as np
import jax
import jax.numpy as jnp
from jax.experimental import pallas as pl
from jax.experimental.pallas import tpu as pltpu

_A = 32
_EPS = 1e-6


def _topk_gates(logits):
    gates = jax.nn.softmax(logits, axis=-1)
    m1 = jnp.max(gates, axis=-1, keepdims=True)
    eq = gates >= m1
    multi = jnp.sum(eq.astype(jnp.float32), axis=-1, keepdims=True) > 1.5
    masked = jnp.where(eq, -1.0, gates)
    m2 = jnp.max(masked, axis=-1, keepdims=True)
    thresh = jnp.where(multi, m1, m2)
    mask = (gates >= thresh).astype(gates.dtype)
    g = gates * mask
    return g / (jnp.sum(g, axis=-1, keepdims=True) + 1e-9)


def _moe_body(*refs, use_act, has_x, num_experts):
    if has_x:
        (h_ref, x_ref, wgh_ref, wgx_ref, bg_ref, weh_ref, wex_ref, be_ref,
         out_ref, g_ref, h16_ref, x16_ref, eo_ref) = refs
    else:
        (h_ref, wgh_ref, bg_ref, weh_ref, be_ref, out_ref, g_ref,
         h16_ref, eo_ref) = refs
        x_ref = wgx_ref = wex_ref = x16_ref = None
    e = pl.program_id(1)

    @pl.when(e == 0)
    def _():
        h16_ref[...] = h_ref[...].astype(jnp.bfloat16)
        if has_x:
            x16_ref[...] = x_ref[...].astype(jnp.bfloat16)
        logits = jnp.dot(h16_ref[...], wgh_ref[...].astype(jnp.bfloat16),
                         preferred_element_type=jnp.float32)
        if has_x:
            logits += jnp.dot(x16_ref[...], wgx_ref[...].astype(jnp.bfloat16),
                              preferred_element_type=jnp.float32)
        logits += bg_ref[...]
        g_ref[...] = _topk_gates(logits)
        out_ref[...] = jnp.zeros_like(out_ref)

    @pl.when(e > 0)
    def _():
        onehot = (jax.lax.broadcasted_iota(jnp.int32, (1, num_experts), 1)
                  == (e - 1)).astype(jnp.float32)
        gcol = jnp.sum(g_ref[...] * onehot, axis=-1, keepdims=True)
        out_ref[...] += (gcol.astype(jnp.bfloat16).astype(jnp.float32)
                         * eo_ref[...].astype(jnp.bfloat16).astype(jnp.float32))

    @pl.when(e < num_experts)
    def _():
        eo = jnp.dot(h16_ref[...], weh_ref[0].astype(jnp.bfloat16),
                     preferred_element_type=jnp.float32)
        if has_x:
            eo += jnp.dot(x16_ref[...], wex_ref[0].astype(jnp.bfloat16),
                          preferred_element_type=jnp.float32)
        eo += be_ref[0]
        if use_act:
            eo = jnp.maximum(eo, 0.2 * eo)
        eo_ref[...] = eo


def _moe_layer(h, x, p, use_act):
    b, dh = h.shape
    E = p['We'].shape[0]
    H = p['We'].shape[-1]
    has_x = x is not None
    wgh = p['Wg'][:dh]
    weh = p['We'][:, :dh]
    bg = p['bg'][None, :]
    be = p['be'][:, None, :]
    bm = b
    inputs = [h]
    specs = [pl.BlockSpec((bm, dh), lambda i, e: (i, 0))]
    if has_x:
        dx = x.shape[1]
        inputs.append(x)
        specs.append(pl.BlockSpec((bm, dx), lambda i, e: (i, 0)))
    inputs.append(wgh)
    specs.append(pl.BlockSpec((dh, E), lambda i, e: (0, 0)))
    if has_x:
        inputs.append(p['Wg'][dh:])
        specs.append(pl.BlockSpec((dx, E), lambda i, e: (0, 0)))
    inputs.append(bg)
    specs.append(pl.BlockSpec((1, E), lambda i, e: (0, 0)))
    wmap = lambda i, e: (jnp.minimum(e, E - 1), 0, 0)
    inputs.append(weh)
    specs.append(pl.BlockSpec((1, dh, H), wmap))
    if has_x:
        inputs.append(p['We'][:, dh:])
        specs.append(pl.BlockSpec((1, dx, H), wmap))
    inputs.append(be)
    specs.append(pl.BlockSpec((1, 1, H), wmap))

    return pl.pallas_call(
        functools.partial(_moe_body, use_act=use_act, has_x=has_x,
                          num_experts=E),
        grid=(b // bm, E + 1),
        in_specs=specs,
        out_specs=pl.BlockSpec((bm, H), lambda i, e: (i, 0)),
        out_shape=jax.ShapeDtypeStruct((b, H), jnp.float32),
        scratch_shapes=(
            [pltpu.VMEM((bm, E), jnp.float32),
             pltpu.VMEM((bm, dh), jnp.bfloat16)]
            + ([pltpu.VMEM((bm, dx), jnp.bfloat16)] if has_x else [])
            + [pltpu.VMEM((bm, H), jnp.float32)]),
        compiler_params=pltpu.CompilerParams(
            dimension_semantics=("arbitrary", "arbitrary")),
    )(*inputs)


def _epi_body(o_ref, n_ref, sq_ref, lp_ref, tm_ref, std_ref):
    out = o_ref[...]
    mean = out[:, :_A]
    log_std = jnp.clip(out[:, _A:], -20.0, 2.0)
    std = jnp.exp(log_std)
    noise = n_ref[...]
    action = mean + noise * std
    squashed = jnp.tanh(action)
    pre = -0.5 * (((action - mean) / (jnp.exp(log_std) + _EPS)) ** 2
                  + 2.0 * log_std + np.log(2.0 * np.pi))
    lp = (jnp.sum(pre, axis=1, keepdims=True)
          - jnp.sum(jnp.log(1.0 - squashed ** 2 + _EPS), axis=1, keepdims=True))
    sq_ref[...] = squashed
    lp_ref[...] = lp
    tm_ref[...] = jnp.tanh(mean)
    std_ref[...] = std


def _epilogue(out, noise):
    b = out.shape[0]
    sq, lp, tm, std = pl.pallas_call(
        _epi_body,
        out_shape=(
            jax.ShapeDtypeStruct((b, _A), jnp.float32),
            jax.ShapeDtypeStruct((b, 1), jnp.float32),
            jax.ShapeDtypeStruct((b, _A), jnp.float32),
            jax.ShapeDtypeStruct((b, _A), jnp.float32),
        ),
    )(out, noise)
    return sq, lp[:, 0], tm, std


def kernel(inp, params):
    h = _moe_layer(inp, None, params['l0'], True)
    h = _moe_layer(h, inp, params['l1'], True)
    h = _moe_layer(h, inp, params['l2'], True)
    h = _moe_layer(h, inp, params['l3'], True)
    out = _moe_layer(h, inp, params['out'], False)
    noise = jax.random.normal(jax.random.key(42), (inp.shape[0], _A),
                              dtype=jnp.float32)
    return _epilogue(out, noise)


# speedup vs baseline: 1.0347x; 1.0347x over previous
import functools
import numpy as np
import jax
import jax.numpy as jnp
from jax.experimental import pallas as pl
from jax.experimental.pallas import tpu as pltpu

_A = 32
_EPS = 1e-6


def _topk_gates(logits):
    gates = jax.nn.softmax(logits, axis=-1)
    m1 = jnp.max(gates, axis=-1, keepdims=True)
    eq = gates >= m1
    multi = jnp.sum(eq.astype(jnp.float32), axis=-1, keepdims=True) > 1.5
    masked = jnp.where(eq, -1.0, gates)
    m2 = jnp.max(masked, axis=-1, keepdims=True)
    thresh = jnp.where(multi, m1, m2)
    mask = (gates >= thresh).astype(gates.dtype)
    g = gates * mask
    return g / (jnp.sum(g, axis=-1, keepdims=True) + 1e-9)


def _moe_body(*refs, use_act, has_x, num_experts):
    if has_x:
        (h_ref, x_ref, wgh_ref, wgx_ref, bg_ref, weh_ref, wex_ref, be_ref,
         out_ref, g_ref, h16_ref, x16_ref, eo_ref) = refs
    else:
        (h_ref, wgh_ref, bg_ref, weh_ref, be_ref, out_ref, g_ref,
         h16_ref, eo_ref) = refs
        x_ref = wgx_ref = wex_ref = x16_ref = None
    e = pl.program_id(1)

    @pl.when(e == 0)
    def _():
        h16_ref[...] = h_ref[...].astype(jnp.bfloat16)
        if has_x:
            x16_ref[...] = x_ref[...].astype(jnp.bfloat16)
        logits = jnp.dot(h16_ref[...], wgh_ref[...].astype(jnp.bfloat16),
                         preferred_element_type=jnp.float32)
        if has_x:
            logits += jnp.dot(x16_ref[...], wgx_ref[...].astype(jnp.bfloat16),
                              preferred_element_type=jnp.float32)
        logits += bg_ref[...]
        g_ref[...] = _topk_gates(logits)
        out_ref[...] = jnp.zeros_like(out_ref)

    @pl.when(e > 0)
    def _():
        onehot = (jax.lax.broadcasted_iota(jnp.int32, (1, num_experts), 1)
                  == (e - 1)).astype(jnp.float32)
        gcol = jnp.sum(g_ref[...] * onehot, axis=-1, keepdims=True)
        out_ref[...] += (gcol.astype(jnp.bfloat16).astype(jnp.float32)
                         * eo_ref[...].astype(jnp.bfloat16).astype(jnp.float32))

    @pl.when(e < num_experts)
    def _():
        eo = jnp.dot(h16_ref[...], weh_ref[0].astype(jnp.bfloat16),
                     preferred_element_type=jnp.float32)
        if has_x:
            eo += jnp.dot(x16_ref[...], wex_ref[0].astype(jnp.bfloat16),
                          preferred_element_type=jnp.float32)
        eo += be_ref[0]
        if use_act:
            eo = jnp.maximum(eo, 0.2 * eo)
        eo_ref[...] = eo


def _moe_layer(h, x, p, use_act):
    b, dh = h.shape
    E = p['We'].shape[0]
    H = p['We'].shape[-1]
    has_x = x is not None
    wgh = p['Wg'][:dh]
    weh = p['We'][:, :dh]
    bg = p['bg'][None, :]
    be = p['be'][:, None, :]
    bm = b
    inputs = [h]
    specs = [pl.BlockSpec((bm, dh), lambda i, e: (i, 0))]
    if has_x:
        dx = x.shape[1]
        inputs.append(x)
        specs.append(pl.BlockSpec((bm, dx), lambda i, e: (i, 0)))
    inputs.append(wgh)
    specs.append(pl.BlockSpec((dh, E), lambda i, e: (0, 0)))
    if has_x:
        inputs.append(p['Wg'][dh:])
        specs.append(pl.BlockSpec((dx, E), lambda i, e: (0, 0)))
    inputs.append(bg)
    specs.append(pl.BlockSpec((1, E), lambda i, e: (0, 0)))
    wmap = lambda i, e: (jnp.minimum(e, E - 1), 0, 0)
    inputs.append(weh)
    specs.append(pl.BlockSpec((1, dh, H), wmap))
    if has_x:
        inputs.append(p['We'][:, dh:])
        specs.append(pl.BlockSpec((1, dx, H), wmap))
    inputs.append(be)
    specs.append(pl.BlockSpec((1, 1, H), wmap))

    return pl.pallas_call(
        functools.partial(_moe_body, use_act=use_act, has_x=has_x,
                          num_experts=E),
        grid=(b // bm, E + 1),
        in_specs=specs,
        out_specs=pl.BlockSpec((bm, H), lambda i, e: (i, 0)),
        out_shape=jax.ShapeDtypeStruct((b, H), jnp.float32),
        scratch_shapes=(
            [pltpu.VMEM((bm, E), jnp.float32),
             pltpu.VMEM((bm, dh), jnp.bfloat16)]
            + ([pltpu.VMEM((bm, dx), jnp.bfloat16)] if has_x else [])
            + [pltpu.VMEM((bm, H), jnp.float32)]),
        compiler_params=pltpu.CompilerParams(
            dimension_semantics=("arbitrary", "arbitrary")),
    )(*inputs)


def _out_epi_body(h_ref, x_ref, wgh_ref, wgx_ref, bg_ref, wet_ref, bef_ref,
                  n_ref, sq_ref, lp_ref, tm_ref, std_ref, *, num_experts):
    h16 = h_ref[...].astype(jnp.bfloat16)
    x16 = x_ref[...].astype(jnp.bfloat16)
    logits = jnp.dot(h16, wgh_ref[...].astype(jnp.bfloat16),
                     preferred_element_type=jnp.float32)
    logits += jnp.dot(x16, wgx_ref[...].astype(jnp.bfloat16),
                      preferred_element_type=jnp.float32)
    logits += bg_ref[...]
    g = _topk_gates(logits)
    wet16 = wet_ref[...].astype(jnp.bfloat16)
    dh = h16.shape[1]
    eo = jnp.dot(h16, wet16[:dh], preferred_element_type=jnp.float32)
    eo += jnp.dot(x16, wet16[dh:], preferred_element_type=jnp.float32)
    eo += bef_ref[...]
    eo16 = eo.astype(jnp.bfloat16).astype(jnp.float32)
    g16 = g.astype(jnp.bfloat16).astype(jnp.float32)
    out = jnp.zeros((h16.shape[0], 2 * _A), jnp.float32)
    for e in range(num_experts):
        out += g16[:, e:e + 1] * eo16[:, e * 2 * _A:(e + 1) * 2 * _A]
    mean = out[:, :_A]
    log_std = jnp.clip(out[:, _A:], -20.0, 2.0)
    std = jnp.exp(log_std)
    noise = n_ref[...]
    action = mean + noise * std
    squashed = jnp.tanh(action)
    pre = -0.5 * (((action - mean) / (jnp.exp(log_std) + _EPS)) ** 2
                  + 2.0 * log_std + np.log(2.0 * np.pi))
    lp = (jnp.sum(pre, axis=1, keepdims=True)
          - jnp.sum(jnp.log(1.0 - squashed ** 2 + _EPS), axis=1, keepdims=True))
    sq_ref[...] = squashed
    lp_ref[...] = lp
    tm_ref[...] = jnp.tanh(mean)
    std_ref[...] = std


def _out_epilogue(h, x, p, noise):
    b, dh = h.shape
    E = p['We'].shape[0]
    wet = jnp.transpose(p['We'], (1, 0, 2)).reshape(dh + x.shape[1],
                                                    E * 2 * _A)
    bef = p['be'].reshape(1, E * 2 * _A)
    sq, lp, tm, std = pl.pallas_call(
        functools.partial(_out_epi_body, num_experts=E),
        out_shape=(
            jax.ShapeDtypeStruct((b, _A), jnp.float32),
            jax.ShapeDtypeStruct((b, 1), jnp.float32),
            jax.ShapeDtypeStruct((b, _A), jnp.float32),
            jax.ShapeDtypeStruct((b, _A), jnp.float32),
        ),
    )(h, x, p['Wg'][:dh], p['Wg'][dh:], p['bg'][None, :], wet, bef, noise)
    return sq, lp[:, 0], tm, std


def kernel(inp, params):
    h = _moe_layer(inp, None, params['l0'], True)
    h = _moe_layer(h, inp, params['l1'], True)
    h = _moe_layer(h, inp, params['l2'], True)
    h = _moe_layer(h, inp, params['l3'], True)
    noise = jax.random.normal(jax.random.key(42), (inp.shape[0], _A),
                              dtype=jnp.float32)
    return _out_epilogue(h, inp, params['out'], noise)


# speedup vs baseline: 1.0500x; 1.0148x over previous
import functools
import numpy as np
import jax
import jax.numpy as jnp
from jax.experimental import pallas as pl
from jax.experimental.pallas import tpu as pltpu

_A = 32
_EPS = 1e-6


def _topk_gates(logits):
    gates = jax.nn.softmax(logits, axis=-1)
    m1 = jnp.max(gates, axis=-1, keepdims=True)
    eq = gates >= m1
    multi = jnp.sum(eq.astype(jnp.float32), axis=-1, keepdims=True) > 1.5
    masked = jnp.where(eq, -1.0, gates)
    m2 = jnp.max(masked, axis=-1, keepdims=True)
    thresh = jnp.where(multi, m1, m2)
    mask = (gates >= thresh).astype(gates.dtype)
    g = gates * mask
    return g / (jnp.sum(g, axis=-1, keepdims=True) + 1e-9)


def _moe_body(*refs, use_act, has_x, num_experts):
    if has_x:
        (h_ref, x_ref, wgh_ref, wgx_ref, bg_ref, weh_ref, wex_ref, be_ref,
         out_ref, g_ref, acc_ref, eo_ref) = refs
    else:
        (h_ref, wgh_ref, bg_ref, weh_ref, be_ref, out_ref, g_ref,
         acc_ref, eo_ref) = refs
        x_ref = wgx_ref = wex_ref = None
    e = pl.program_id(1)

    @pl.when(e == 0)
    def _():
        logits = jnp.dot(h_ref[...], wgh_ref[...].astype(jnp.bfloat16),
                         preferred_element_type=jnp.float32)
        if has_x:
            logits += jnp.dot(x_ref[...], wgx_ref[...].astype(jnp.bfloat16),
                              preferred_element_type=jnp.float32)
        logits += bg_ref[...]
        g_ref[...] = _topk_gates(logits)
        acc_ref[...] = jnp.zeros_like(acc_ref)

    @pl.when(e > 0)
    def _():
        onehot = (jax.lax.broadcasted_iota(jnp.int32, (1, num_experts), 1)
                  == (e - 1)).astype(jnp.float32)
        gcol = jnp.sum(g_ref[...] * onehot, axis=-1, keepdims=True)
        acc_ref[...] += (gcol.astype(jnp.bfloat16).astype(jnp.float32)
                         * eo_ref[...].astype(jnp.bfloat16).astype(jnp.float32))

    @pl.when(e < num_experts)
    def _():
        eo = jnp.dot(h_ref[...], weh_ref[0].astype(jnp.bfloat16),
                     preferred_element_type=jnp.float32)
        if has_x:
            eo += jnp.dot(x_ref[...], wex_ref[0].astype(jnp.bfloat16),
                          preferred_element_type=jnp.float32)
        eo += be_ref[0]
        if use_act:
            eo = jnp.maximum(eo, 0.2 * eo)
        eo_ref[...] = eo

    @pl.when(e == num_experts)
    def _():
        out_ref[...] = acc_ref[...].astype(jnp.bfloat16)


def _moe_layer(h, x, p, use_act):
    b, dh = h.shape
    E = p['We'].shape[0]
    H = p['We'].shape[-1]
    has_x = x is not None
    wgh = p['Wg'][:dh]
    weh = p['We'][:, :dh]
    bg = p['bg'][None, :]
    be = p['be'][:, None, :]
    bm = b
    inputs = [h]
    specs = [pl.BlockSpec((bm, dh), lambda i, e: (i, 0))]
    if has_x:
        dx = x.shape[1]
        inputs.append(x)
        specs.append(pl.BlockSpec((bm, dx), lambda i, e: (i, 0)))
    inputs.append(wgh)
    specs.append(pl.BlockSpec((dh, E), lambda i, e: (0, 0)))
    if has_x:
        inputs.append(p['Wg'][dh:])
        specs.append(pl.BlockSpec((dx, E), lambda i, e: (0, 0)))
    inputs.append(bg)
    specs.append(pl.BlockSpec((1, E), lambda i, e: (0, 0)))
    wmap = lambda i, e: (jnp.minimum(e, E - 1), 0, 0)
    inputs.append(weh)
    specs.append(pl.BlockSpec((1, dh, H), wmap))
    if has_x:
        inputs.append(p['We'][:, dh:])
        specs.append(pl.BlockSpec((1, dx, H), wmap))
    inputs.append(be)
    specs.append(pl.BlockSpec((1, 1, H), wmap))

    return pl.pallas_call(
        functools.partial(_moe_body, use_act=use_act, has_x=has_x,
                          num_experts=E),
        grid=(b // bm, E + 1),
        in_specs=specs,
        out_specs=pl.BlockSpec((bm, H), lambda i, e: (i, 0)),
        out_shape=jax.ShapeDtypeStruct((b, H), jnp.bfloat16),
        scratch_shapes=[pltpu.VMEM((bm, E), jnp.float32),
                        pltpu.VMEM((bm, H), jnp.float32),
                        pltpu.VMEM((bm, H), jnp.float32)],
        compiler_params=pltpu.CompilerParams(
            dimension_semantics=("arbitrary", "arbitrary")),
    )(*inputs)


def _out_epi_body(h_ref, x_ref, wgh_ref, wgx_ref, bg_ref, wet_ref, bef_ref,
                  n_ref, sq_ref, lp_ref, tm_ref, std_ref, *, num_experts):
    h16 = h_ref[...]
    x16 = x_ref[...]
    logits = jnp.dot(h16, wgh_ref[...].astype(jnp.bfloat16),
                     preferred_element_type=jnp.float32)
    logits += jnp.dot(x16, wgx_ref[...].astype(jnp.bfloat16),
                      preferred_element_type=jnp.float32)
    logits += bg_ref[...]
    g = _topk_gates(logits)
    wet16 = wet_ref[...].astype(jnp.bfloat16)
    dh = h16.shape[1]
    eo = jnp.dot(h16, wet16[:dh], preferred_element_type=jnp.float32)
    eo += jnp.dot(x16, wet16[dh:], preferred_element_type=jnp.float32)
    eo += bef_ref[...]
    eo16 = eo.astype(jnp.bfloat16).astype(jnp.float32)
    g16 = g.astype(jnp.bfloat16).astype(jnp.float32)
    out = jnp.zeros((h16.shape[0], 2 * _A), jnp.float32)
    for e in range(num_experts):
        out += g16[:, e:e + 1] * eo16[:, e * 2 * _A:(e + 1) * 2 * _A]
    mean = out[:, :_A]
    log_std = jnp.clip(out[:, _A:], -20.0, 2.0)
    std = jnp.exp(log_std)
    noise = n_ref[...]
    action = mean + noise * std
    squashed = jnp.tanh(action)
    pre = -0.5 * (((action - mean) / (jnp.exp(log_std) + _EPS)) ** 2
                  + 2.0 * log_std + np.log(2.0 * np.pi))
    lp = (jnp.sum(pre, axis=1, keepdims=True)
          - jnp.sum(jnp.log(1.0 - squashed ** 2 + _EPS), axis=1, keepdims=True))
    sq_ref[...] = squashed
    lp_ref[...] = lp
    tm_ref[...] = jnp.tanh(mean)
    std_ref[...] = std


def _out_epilogue(h, x, p, noise):
    b, dh = h.shape
    E = p['We'].shape[0]
    wet = jnp.transpose(p['We'], (1, 0, 2)).reshape(dh + x.shape[1],
                                                    E * 2 * _A)
    bef = p['be'].reshape(1, E * 2 * _A)
    sq, lp, tm, std = pl.pallas_call(
        functools.partial(_out_epi_body, num_experts=E),
        out_shape=(
            jax.ShapeDtypeStruct((b, _A), jnp.float32),
            jax.ShapeDtypeStruct((b, 1), jnp.float32),
            jax.ShapeDtypeStruct((b, _A), jnp.float32),
            jax.ShapeDtypeStruct((b, _A), jnp.float32),
        ),
    )(h, x, p['Wg'][:dh], p['Wg'][dh:], p['bg'][None, :], wet, bef, noise)
    return sq, lp[:, 0], tm, std


def kernel(inp, params):
    x16 = inp.astype(jnp.bfloat16)
    h = _moe_layer(x16, None, params['l0'], True)
    h = _moe_layer(h, x16, params['l1'], True)
    h = _moe_layer(h, x16, params['l2'], True)
    h = _moe_layer(h, x16, params['l3'], True)
    noise = jax.random.normal(jax.random.key(42), (inp.shape[0], _A),
                              dtype=jnp.float32)
    return _out_epilogue(h, x16, params['out'], noise)


# speedup vs baseline: 1.0546x; 1.0044x over previous
import functools
import numpy as np
import jax
import jax.numpy as jnp
from jax.experimental import pallas as pl
from jax.experimental.pallas import tpu as pltpu

_A = 32
_EPS = 1e-6


def _topk_gates(logits):
    gates = jax.nn.softmax(logits, axis=-1)
    m1 = jnp.max(gates, axis=-1, keepdims=True)
    eq = gates >= m1
    multi = jnp.sum(eq.astype(jnp.float32), axis=-1, keepdims=True) > 1.5
    masked = jnp.where(eq, -1.0, gates)
    m2 = jnp.max(masked, axis=-1, keepdims=True)
    thresh = jnp.where(multi, m1, m2)
    mask = (gates >= thresh).astype(gates.dtype)
    g = gates * mask
    return g / (jnp.sum(g, axis=-1, keepdims=True) + 1e-9)


def _moe_body(*refs, use_act, has_x, num_experts):
    if has_x:
        (h_ref, x_ref, wgh_ref, wgx_ref, bg_ref, weh_ref, wex_ref, be_ref,
         out_ref, g_ref, acc_ref, eo_ref) = refs
    else:
        (h_ref, wgh_ref, bg_ref, weh_ref, be_ref, out_ref, g_ref,
         acc_ref, eo_ref) = refs
        x_ref = wgx_ref = wex_ref = None
    e = pl.program_id(1)

    @pl.when(e == 0)
    def _():
        logits = jnp.dot(h_ref[...], wgh_ref[...].astype(jnp.bfloat16),
                         preferred_element_type=jnp.float32)
        if has_x:
            logits += jnp.dot(x_ref[...], wgx_ref[...].astype(jnp.bfloat16),
                              preferred_element_type=jnp.float32)
        logits += bg_ref[...]
        g_ref[...] = _topk_gates(logits)
        acc_ref[...] = jnp.zeros_like(acc_ref)

    @pl.when(e > 0)
    def _():
        onehot = (jax.lax.broadcasted_iota(jnp.int32, (1, num_experts), 1)
                  == (e - 1)).astype(jnp.float32)
        gcol = jnp.sum(g_ref[...] * onehot, axis=-1, keepdims=True)
        acc_ref[...] += (gcol.astype(jnp.bfloat16).astype(jnp.float32)
                         * eo_ref[...].astype(jnp.float32))

    @pl.when(e < num_experts)
    def _():
        eo = jnp.dot(h_ref[...], weh_ref[0].astype(jnp.bfloat16),
                     preferred_element_type=jnp.float32)
        if has_x:
            eo += jnp.dot(x_ref[...], wex_ref[0].astype(jnp.bfloat16),
                          preferred_element_type=jnp.float32)
        eo += be_ref[0]
        if use_act:
            eo = jnp.maximum(eo, 0.2 * eo)
        eo_ref[...] = eo.astype(jnp.bfloat16)

    @pl.when(e == num_experts)
    def _():
        out_ref[...] = acc_ref[...].astype(jnp.bfloat16)


def _moe_layer(h, x, p, use_act):
    b, dh = h.shape
    E = p['We'].shape[0]
    H = p['We'].shape[-1]
    has_x = x is not None
    wgh = p['Wg'][:dh]
    weh = p['We'][:, :dh]
    bg = p['bg'][None, :]
    be = p['be'][:, None, :]
    bm = b
    inputs = [h]
    specs = [pl.BlockSpec((bm, dh), lambda i, e: (i, 0))]
    if has_x:
        dx = x.shape[1]
        inputs.append(x)
        specs.append(pl.BlockSpec((bm, dx), lambda i, e: (i, 0)))
    inputs.append(wgh)
    specs.append(pl.BlockSpec((dh, E), lambda i, e: (0, 0)))
    if has_x:
        inputs.append(p['Wg'][dh:])
        specs.append(pl.BlockSpec((dx, E), lambda i, e: (0, 0)))
    inputs.append(bg)
    specs.append(pl.BlockSpec((1, E), lambda i, e: (0, 0)))
    wmap = lambda i, e: (jnp.minimum(e, E - 1), 0, 0)
    inputs.append(weh)
    specs.append(pl.BlockSpec((1, dh, H), wmap))
    if has_x:
        inputs.append(p['We'][:, dh:])
        specs.append(pl.BlockSpec((1, dx, H), wmap))
    inputs.append(be)
    specs.append(pl.BlockSpec((1, 1, H), wmap))

    return pl.pallas_call(
        functools.partial(_moe_body, use_act=use_act, has_x=has_x,
                          num_experts=E),
        grid=(b // bm, E + 1),
        in_specs=specs,
        out_specs=pl.BlockSpec((bm, H), lambda i, e: (i, 0)),
        out_shape=jax.ShapeDtypeStruct((b, H), jnp.bfloat16),
        scratch_shapes=[pltpu.VMEM((bm, E), jnp.float32),
                        pltpu.VMEM((bm, H), jnp.float32),
                        pltpu.VMEM((bm, H), jnp.bfloat16)],
        compiler_params=pltpu.CompilerParams(
            dimension_semantics=("arbitrary", "arbitrary")),
    )(*inputs)


def _out_epi_body(h_ref, x_ref, wgh_ref, wgx_ref, bg_ref, wet_ref, bef_ref,
                  n_ref, sq_ref, lp_ref, tm_ref, std_ref, *, num_experts):
    h16 = h_ref[...]
    x16 = x_ref[...]
    logits = jnp.dot(h16, wgh_ref[...].astype(jnp.bfloat16),
                     preferred_element_type=jnp.float32)
    logits += jnp.dot(x16, wgx_ref[...].astype(jnp.bfloat16),
                      preferred_element_type=jnp.float32)
    logits += bg_ref[...]
    g = _topk_gates(logits)
    wet16 = wet_ref[...].astype(jnp.bfloat16)
    dh = h16.shape[1]
    eo = jnp.dot(h16, wet16[:dh], preferred_element_type=jnp.float32)
    eo += jnp.dot(x16, wet16[dh:], preferred_element_type=jnp.float32)
    eo += bef_ref[...]
    eo16 = eo.astype(jnp.bfloat16).astype(jnp.float32)
    g16 = g.astype(jnp.bfloat16).astype(jnp.float32)
    out = jnp.zeros((h16.shape[0], 2 * _A), jnp.float32)
    for e in range(num_experts):
        out += g16[:, e:e + 1] * eo16[:, e * 2 * _A:(e + 1) * 2 * _A]
    mean = out[:, :_A]
    log_std = jnp.clip(out[:, _A:], -20.0, 2.0)
    std = jnp.exp(log_std)
    noise = n_ref[...]
    action = mean + noise * std
    squashed = jnp.tanh(action)
    pre = -0.5 * (((action - mean) / (jnp.exp(log_std) + _EPS)) ** 2
                  + 2.0 * log_std + np.log(2.0 * np.pi))
    lp = (jnp.sum(pre, axis=1, keepdims=True)
          - jnp.sum(jnp.log(1.0 - squashed ** 2 + _EPS), axis=1, keepdims=True))
    sq_ref[...] = squashed
    lp_ref[...] = lp
    tm_ref[...] = jnp.tanh(mean)
    std_ref[...] = std


def _out_epilogue(h, x, p, noise):
    b, dh = h.shape
    E = p['We'].shape[0]
    wet = jnp.transpose(p['We'], (1, 0, 2)).reshape(dh + x.shape[1],
                                                    E * 2 * _A)
    bef = p['be'].reshape(1, E * 2 * _A)
    sq, lp, tm, std = pl.pallas_call(
        functools.partial(_out_epi_body, num_experts=E),
        out_shape=(
            jax.ShapeDtypeStruct((b, _A), jnp.float32),
            jax.ShapeDtypeStruct((b, 1), jnp.float32),
            jax.ShapeDtypeStruct((b, _A), jnp.float32),
            jax.ShapeDtypeStruct((b, _A), jnp.float32),
        ),
    )(h, x, p['Wg'][:dh], p['Wg'][dh:], p['bg'][None, :], wet, bef, noise)
    return sq, lp[:, 0], tm, std


def kernel(inp, params):
    x16 = inp.astype(jnp.bfloat16)
    h = _moe_layer(x16, None, params['l0'], True)
    h = _moe_layer(h, x16, params['l1'], True)
    h = _moe_layer(h, x16, params['l2'], True)
    h = _moe_layer(h, x16, params['l3'], True)
    noise = jax.random.normal(jax.random.key(42), (inp.shape[0], _A),
                              dtype=jnp.float32)
    return _out_epilogue(h, x16, params['out'], noise)
